# Initial kernel scaffold; baseline (speedup 1.0000x reference)
import jax
import jax.numpy as jnp
from jax.experimental import pallas as pl


def kernel(xyz, params):
    raise NotImplementedError("write your pallas kernel here")


# speedup vs baseline: 5.2782x; 5.2782x over previous
import functools

import jax
import jax.numpy as jnp
import numpy as np
from jax.experimental import pallas as pl

_NPOINT1, _NPOINT2 = 512, 128
_RADII1 = [0.1, 0.2, 0.4]
_RADII2 = [0.2, 0.4, 0.8]
_NS1 = [16, 32, 128]
_NS2 = [32, 64, 128]
_BN_SCALE = 1.0 / np.sqrt(1.0 + 1e-5)

_F32 = jnp.float32


def _prep(layers):
    out = []
    for (W, b, g, be) in layers:
        s = g * _BN_SCALE
        out.append((W, (s).reshape(1, -1), (b * s + be).reshape(1, -1)))
    return out


def _fps_kernel(npoint, xt_ref, cx_ref, cy_ref, cz_ref):
    X = xt_ref[0]
    Y = xt_ref[1]
    Z = xt_ref[2]
    B, N = X.shape
    iota = jax.lax.broadcasted_iota(jnp.int32, (B, N), 1)
    iota_s = jax.lax.broadcasted_iota(jnp.int32, (B, npoint), 1)

    def body(i, carry):
        dist, far, acx, acy, acz = carry
        sel = iota == far
        cx = jnp.sum(jnp.where(sel, X, 0.0), axis=1, keepdims=True)
        cy = jnp.sum(jnp.where(sel, Y, 0.0), axis=1, keepdims=True)
        cz = jnp.sum(jnp.where(sel, Z, 0.0), axis=1, keepdims=True)
        hit = iota_s == i
        acx = jnp.where(hit, cx, acx)
        acy = jnp.where(hit, cy, acy)
        acz = jnp.where(hit, cz, acz)
        d = (X - cx) ** 2 + (Y - cy) ** 2 + (Z - cz) ** 2
        dist = jnp.minimum(dist, d)
        m = jnp.max(dist, axis=1, keepdims=True)
        far = jnp.min(jnp.where(dist == m, iota, N), axis=1, keepdims=True)
        return dist, far, acx, acy, acz

    z = jnp.zeros((B, npoint), _F32)
    _, _, acx, acy, acz = jax.lax.fori_loop(
        0, npoint, body,
        (jnp.full((B, N), 1e10, _F32), jnp.zeros((B, 1), jnp.int32), z, z, z))
    cx_ref[...] = acx
    cy_ref[...] = acy
    cz_ref[...] = acz


def _fps(xt, npoint):
    _, B, N = xt.shape
    out = jax.ShapeDtypeStruct((B, npoint), _F32)
    return pl.pallas_call(
        functools.partial(_fps_kernel, npoint),
        out_shape=[out, out, out],
    )(xt)


def _branch_kernel(r2, K, S_T, xyz_col, gprec, ptsT_ref, F_ref, nxyz_ref,
                   tri_ref, *refs):
    ws = refs[:-1]
    out_ref = refs[-1]
    ptsT = ptsT_ref[0]
    F = F_ref[0]
    C = nxyz_ref[0]
    N = ptsT.shape[1]
    CF = F.shape[1]

    d_sq = jnp.sum(ptsT * ptsT, axis=0, keepdims=True)
    s_sq = jnp.sum(C * C, axis=1, keepdims=True)
    dot = jnp.dot(C, ptsT, preferred_element_type=_F32)
    sqr = s_sq + d_sq - 2.0 * dot

    mask = (sqr <= r2).astype(_F32)
    rank_incl = jnp.dot(mask.astype(jnp.bfloat16), tri_ref[...],
                        preferred_element_type=_F32)
    rank = rank_incl - mask
    P = jnp.where(mask > 0.0, rank, -1.0).astype(jnp.int32)
    cnt = jnp.sum(mask, axis=1, keepdims=True).astype(jnp.int32)
    cnt2 = cnt + jnp.zeros((S_T, N), jnp.int32)

    iota_n = jax.lax.broadcasted_iota(jnp.int32, (S_T, N), 1)
    P = jnp.where((cnt2 == 0) & (iota_n == N - 1), 0, P)

    kio = jax.lax.broadcasted_iota(jnp.int32, (S_T, K, N), 1)
    kc = jnp.maximum(jnp.minimum(kio, cnt2[:, None, :] - 1), 0)
    oh = (P[:, None, :] == kc).astype(_F32)
    G = jnp.dot(oh.reshape(S_T * K, N), F,
                preferred_element_type=_F32, precision=gprec)

    ctr = jnp.concatenate(
        [jnp.zeros((S_T, xyz_col), _F32), C[:, :3],
         jnp.zeros((S_T, CF - xyz_col - 3), _F32)], axis=1)
    x = (G.reshape(S_T, K, CF) - ctr[:, None, :]).reshape(S_T * K, CF)

    for i in range(0, len(ws), 3):
        W = ws[i][...]
        sc = ws[i + 1][...]
        bi = ws[i + 2][...]
        x = jnp.maximum(
            jnp.dot(x, W, preferred_element_type=_F32) * sc + bi, 0.0)

    Cout = x.shape[1]
    out_ref[0] = jnp.max(x.reshape(S_T, K, Cout), axis=1)


def _sa_branch(ptsT, F, nxyz, tri, weights, r2, K, S_T, xyz_col, gprec):
    B, _, N = ptsT.shape
    S = nxyz.shape[1]
    CF = F.shape[2]
    Cout = weights[-1][0].shape[1]
    flat_w = []
    w_specs = []
    for (W, sc, bi) in weights:
        flat_w += [W, sc, bi]
        w_specs.append(pl.BlockSpec(W.shape, lambda b_, s_: (0, 0)))
        w_specs.append(pl.BlockSpec(sc.shape, lambda b_, s_: (0, 0)))
        w_specs.append(pl.BlockSpec(bi.shape, lambda b_, s_: (0, 0)))
    return pl.pallas_call(
        functools.partial(_branch_kernel, r2, K, S_T, xyz_col, gprec),
        grid=(B, S // S_T),
        in_specs=[
            pl.BlockSpec((1, 8, N), lambda b_, s_: (b_, 0, 0)),
            pl.BlockSpec((1, N, CF), lambda b_, s_: (b_, 0, 0)),
            pl.BlockSpec((1, S_T, 8), lambda b_, s_: (b_, s_, 0)),
            pl.BlockSpec((N, N), lambda b_, s_: (0, 0)),
        ] + w_specs,
        out_specs=pl.BlockSpec((1, S_T, Cout), lambda b_, s_: (b_, s_, 0)),
        out_shape=jax.ShapeDtypeStruct((B, S, Cout), _F32),
    )(ptsT, F, nxyz, tri, *flat_w)


def _head_kernel(n_mlp, cat_ref, *refs):
    ws = refs[:-1]
    out_ref = refs[-1]
    x = cat_ref[0]
    i = 0
    for _ in range(n_mlp):
        x = jnp.maximum(
            jnp.dot(x, ws[i][...], preferred_element_type=_F32)
            * ws[i + 1][...] + ws[i + 2][...], 0.0)
        i += 3
    x = jnp.max(x, axis=0, keepdims=True)
    for _ in range(2):
        x = jnp.maximum(
            jnp.dot(x, ws[i][...], preferred_element_type=_F32)
            * ws[i + 1][...] + ws[i + 2][...], 0.0)
        i += 3
    out_ref[0] = (jnp.dot(x, ws[i][...], preferred_element_type=_F32)
                  * ws[i + 1][...] + ws[i + 2][...])


def _head(cat, mlp_w, fc_w):
    B, S, Cc = cat.shape
    all_w = list(mlp_w) + list(fc_w)
    flat_w = []
    w_specs = []
    for (W, sc, bi) in all_w:
        flat_w += [W, sc, bi]
        w_specs.append(pl.BlockSpec(W.shape, lambda b_: (0, 0)))
        w_specs.append(pl.BlockSpec(sc.shape, lambda b_: (0, 0)))
        w_specs.append(pl.BlockSpec(bi.shape, lambda b_: (0, 0)))
    return pl.pallas_call(
        functools.partial(_head_kernel, len(mlp_w)),
        grid=(B,),
        in_specs=[pl.BlockSpec((1, S, Cc), lambda b_: (b_, 0, 0))] + w_specs,
        out_specs=pl.BlockSpec((1, 1, 128), lambda b_: (b_, 0, 0)),
        out_shape=jax.ShapeDtypeStruct((B, 1, 128), _F32),
    )(cat, *flat_w)


def _pad_rows(W, rows):
    return jnp.pad(W, ((0, rows - W.shape[0]), (0, 0)))


def _sa_layer(ptsT, F, cxyz, tri, branch_params, radii, nsamples, s_tiles,
              xyz_col, gprec):
    B = ptsT.shape[0]
    S = cxyz[0].shape[1]
    nxyz = jnp.stack(list(cxyz) + [jnp.zeros_like(cxyz[0])] * 5, axis=2)
    outs = []
    for (r, K, layers, S_T) in zip(radii, nsamples, branch_params, s_tiles):
        prep = _prep(layers)
        W1, sc1, bi1 = prep[0]
        prep = [(_pad_rows(W1, F.shape[2]), sc1, bi1)] + prep[1:]
        outs.append(_sa_branch(ptsT, F, nxyz, tri, prep, r * r, K, S_T,
                               xyz_col, gprec))
    return jnp.concatenate(outs, axis=-1)


def kernel(xyz, params):
    B, _, N = xyz.shape
    xyzc = jnp.transpose(xyz, (0, 2, 1))

    tri1 = (jax.lax.broadcasted_iota(jnp.int32, (N, N), 0)
            <= jax.lax.broadcasted_iota(jnp.int32, (N, N), 1)
            ).astype(jnp.bfloat16)
    tri2 = tri1[:_NPOINT1, :_NPOINT1]

    xt1 = jnp.transpose(xyz[:, :3, :], (1, 0, 2))
    cx1, cy1, cz1 = _fps(xt1, _NPOINT1)
    ptsT1 = jnp.concatenate(
        [xyz[:, :3, :], jnp.zeros((B, 5, N), _F32)], axis=1)
    F1 = jnp.concatenate(
        [xyzc[:, :, 3:6], xyzc[:, :, 0:3], jnp.zeros((B, N, 2), _F32)],
        axis=2)
    l1_points = _sa_layer(ptsT1, F1, (cx1, cy1, cz1), tri1,
                          params['sa1'], _RADII1, _NS1, (32, 16, 8),
                          xyz_col=3, gprec=jax.lax.Precision.HIGHEST)

    xt2 = jnp.stack([cx1, cy1, cz1])
    cx2, cy2, cz2 = _fps(xt2, _NPOINT2)
    ptsT2 = jnp.concatenate(
        [xt2.transpose(1, 0, 2), jnp.zeros((B, 5, _NPOINT1), _F32)], axis=1)
    l1_xyz = jnp.stack([cx1, cy1, cz1], axis=2)
    F2 = jnp.concatenate(
        [l1_points, l1_xyz, jnp.zeros((B, _NPOINT1, 61), _F32)],
        axis=2)
    l2_points = _sa_layer(ptsT2, F2, (cx2, cy2, cz2), tri2,
                          params['sa2'], _RADII2, _NS2, (16, 16, 16),
                          xyz_col=320, gprec=jax.lax.Precision.HIGHEST)

    l2_xyz = jnp.stack([cx2, cy2, cz2], axis=2)
    cat = jnp.concatenate(
        [l2_xyz, l2_points, jnp.zeros((B, _NPOINT2, 5), _F32)], axis=2)
    mlp3 = _prep(params['sa3'])
    mlp3 = [(_pad_rows(mlp3[0][0], cat.shape[2]),) + mlp3[0][1:]] + mlp3[1:]
    fc1 = _prep([params['fc1']])[0]
    fc2 = _prep([params['fc2']])[0]
    W3, b3 = params['fc3']
    W3p = jnp.pad(W3, ((0, 0), (0, 128 - W3.shape[1])))
    b3p = jnp.pad(b3, (0, 128 - b3.shape[0])).reshape(1, -1)
    ones = jnp.ones((1, 128), _F32)
    out = _head(cat, mlp3, [fc1, fc2, (W3p, ones, b3p)])
    return out[:, 0, :27]
